# Initial kernel scaffold; baseline (speedup 1.0000x reference)
import jax
import jax.numpy as jnp
from jax.experimental import pallas as pl


def kernel(symbols_prob):
    raise NotImplementedError("write your pallas kernel here")


# speedup vs baseline: 323.3983x; 323.3983x over previous
import jax
import jax.numpy as jnp
import numpy as np
from jax.experimental import pallas as pl
from jax.experimental.pallas import tpu as pltpu

_N_STATES = 16
_N_SYMBOLS = 8
_SEQ_LEN = 200

_DFA = {0: {'0': 1, '1': 2, 'and(2,3)': 3}, 1: {'2': 3, 'not(0)': 0, '4': 5}, 2: {'or(1,5)': 4, '3': 2}, 3: {'5': 6, 'T': 0}, 4: {'6': 7, 'and(0,not(1))': 8}, 5: {'7': 9, '2': 5}, 6: {'or(and(0,1),2)': 10, '4': 6}, 7: {'1': 11, 'not(6)': 7}, 8: {'3': 12, '0': 8}, 9: {'5': 13, 'or(2,3)': 9}, 10: {'and(4,5)': 14, '6': 10}, 11: {'7': 15, '1': 11}, 12: {'0': 0, 'not(7)': 12}, 13: {'2': 1, '6': 13}, 14: {'or(0,not(4))': 2, '3': 14}, 15: {'T': 3}}

_TRANS = [(s, g, d) for s in sorted(_DFA.keys()) for g, d in _DFA[s].items()]


def _divide_args(guard):
    args = guard.split(',')
    out = []
    i = 0
    while i < len(args):
        a = args[i]
        while a.count('(') != a.count(')'):
            i += 1
            a = a + ',' + args[i]
        out.append(a)
        i += 1
    return out


def _eval_guard(guard, cols):
    if guard[0] == 'a':
        v = 1.0
        for a in _divide_args(guard[4:-1]):
            v = v * _eval_guard(a, cols)
        return v
    elif guard[0] == 'o':
        v = 0.0
        for a in _divide_args(guard[3:-1]):
            e = _eval_guard(a, cols)
            v = v + e - v * e
        return v
    elif guard[0] == 'n':
        return 1.0 - _eval_guard(guard[4:-1], cols)
    elif guard[0] == 'T':
        return jnp.ones_like(cols[0])
    else:
        return cols[int(guard)]


_EDGE = {(d, s): t for t, (s, _, d) in enumerate(_TRANS)}


def _build_mats(p_block):
    L = p_block.shape[0]
    cols = [p_block[:, k:k + 1] for k in range(_N_SYMBOLS)]
    gvals = [_eval_guard(g, cols) for (_, g, _) in _TRANS]
    zero = jnp.zeros((L, 1), dtype=p_block.dtype)

    def stack2d(index_fn):
        rows = []
        for a in range(_N_STATES):
            row = [index_fn(a, b) for b in range(_N_STATES)]
            rows.append(jnp.concatenate(row, axis=1)[:, None, :])
        return jnp.concatenate(rows, axis=1)

    mats_ds = stack2d(lambda d, s: gvals[_EDGE[(d, s)]] if (d, s) in _EDGE else zero)
    mats_sd = stack2d(lambda s, d: gvals[_EDGE[(d, s)]] if (d, s) in _EDGE else zero)
    return mats_ds, mats_sd


def _fuzzy_kernel(p_ref, out_ref, ads_ref, asd_ref):
    p = p_ref[:, :].reshape(_SEQ_LEN // 2, 2, _N_SYMBOLS)
    p_even = p[:, 0, :]
    p_odd = p[:, 1, :]

    ads_ref[:, :, :], _ = _build_mats(p_even)
    _, asd = _build_mats(p_odd)
    asd_ref[:, :, :] = asd

    st0 = (jax.lax.broadcasted_iota(jnp.int32, (1, _N_STATES), 1) == 0
           ).astype(p_ref.dtype)

    def body(i, st):
        a = ads_ref[i]
        mid = jnp.sum(a * st, axis=1, keepdims=True)
        b = asd_ref[i]
        return jnp.sum(b * mid, axis=0, keepdims=True)

    st = jax.lax.fori_loop(0, _SEQ_LEN // 2, body, st0)
    out_ref[:, :] = st


def kernel(symbols_prob):
    out = pl.pallas_call(
        _fuzzy_kernel,
        out_shape=jax.ShapeDtypeStruct((1, _N_STATES), symbols_prob.dtype),
        scratch_shapes=[
            pltpu.VMEM((_SEQ_LEN // 2, _N_STATES, _N_STATES), symbols_prob.dtype),
            pltpu.VMEM((_SEQ_LEN // 2, _N_STATES, _N_STATES), symbols_prob.dtype),
        ],
    )(symbols_prob)
    return out.reshape(_N_STATES)
